# Initial kernel scaffold; baseline (speedup 1.0000x reference)
import jax
import jax.numpy as jnp
from jax.experimental import pallas as pl


def kernel(features, edge_index, W, b):
    raise NotImplementedError("write your pallas kernel here")


# speedup vs baseline: 2.9153x; 2.9153x over previous
import dataclasses

import jax
import jax.numpy as jnp
from jax import lax
from jax.experimental import pallas as pl
from jax.experimental.pallas import tpu as pltpu
from jax.experimental.pallas import tpu_sc as plsc

N_NODES = 10000
N_HEDGES = 10000
N_EDGES = 320000
D = 128

NC = 2
NS = 16
NW = NC * NS
CHUNK = 128
CPT = 80
E_PAD = NW * CPT * CHUNK
SEG = 10240
PAD_ROW = 10200
ROWS_PER_TILE = SEG // NS

_mesh = plsc.VectorSubcoreMesh(
    core_axis_name="c", subcore_axis_name="s", num_cores=NC, num_subcores=NS
)


def _zero_vmem(ref):
    rows, cols = ref.shape

    @pl.loop(0, rows)
    def _(i):
        for j in range(cols // 16):
            ref[i, pl.ds(j * 16, 16)] = jnp.zeros((16,), ref.dtype)


def _make_pass():
    outs = (
        jax.ShapeDtypeStruct((NC, SEG, D), jnp.float32),
        jax.ShapeDtypeStruct((NW, SEG), jnp.float32),
    )
    scratch = [
        pltpu.VMEM_SHARED((SEG, D), jnp.float32),
        pltpu.VMEM((CHUNK, D), jnp.float32),
        pltpu.VMEM((SEG,), jnp.float32),
        pltpu.VMEM((CHUNK,), jnp.int32),
        pltpu.VMEM((CHUNK,), jnp.int32),
        pltpu.SemaphoreType.DMA,
    ]

    def body(table_hbm, gidx_hbm, sidx_hbm, sum_out, cnt_out,
             sum_sh, rows_v, hist_v, gbuf_v, sbuf_v, sem):
        c = lax.axis_index("c")
        s = lax.axis_index("s")
        w = c * NS + s

        _zero_vmem(rows_v)
        rbase = s * ROWS_PER_TILE
        for t in range(ROWS_PER_TILE // CHUNK):
            pltpu.sync_copy(rows_v, sum_sh.at[pl.ds(rbase + t * CHUNK, CHUNK)])

        @pl.loop(0, SEG // 16)
        def _(i):
            hist_v[pl.ds(i * 16, 16)] = jnp.zeros((16,), jnp.float32)

        plsc.subcore_barrier()

        cbase = w * CPT * CHUNK

        @pl.loop(0, CPT)
        def _(k):
            eb = cbase + k * CHUNK
            pltpu.sync_copy(gidx_hbm.at[pl.ds(eb, CHUNK)], gbuf_v)
            pltpu.sync_copy(sidx_hbm.at[pl.ds(eb, CHUNK)], sbuf_v)
            pltpu.async_copy(table_hbm.at[gbuf_v], rows_v, sem).wait()
            pltpu.sync_copy(rows_v, sum_sh.at[sbuf_v], add=True)

            @pl.loop(0, CHUNK // 16)
            def _(q):
                iv = sbuf_v[pl.ds(q * 16, 16)]
                lane = lax.iota(jnp.int32, 16)
                counts = jnp.zeros((16,), jnp.float32)
                firstpos = jnp.full((16,), 16, jnp.int32)
                for j in range(15, -1, -1):
                    eq = iv == iv[j]
                    counts = counts + jnp.where(eq, 1.0, 0.0)
                    firstpos = jnp.where(eq, j, firstpos)
                head = firstpos == lane
                plsc.addupdate_scatter(hist_v, [iv], counts, mask=head)

        plsc.subcore_barrier()

        for t in range(ROWS_PER_TILE // CHUNK):
            sl = pl.ds(rbase + t * CHUNK, CHUNK)
            pltpu.sync_copy(sum_sh.at[sl], rows_v)
            pltpu.sync_copy(rows_v, sum_out.at[c, sl])

        pltpu.sync_copy(hist_v, cnt_out.at[w])

    cp = pltpu.CompilerParams()
    if "needs_layout_passes" in pltpu.CompilerParams.__dataclass_fields__:
        cp = dataclasses.replace(cp, needs_layout_passes=False)
    return pl.kernel(
        body, out_type=outs, mesh=_mesh, scratch_types=scratch,
        compiler_params=cp)


_scatter_mean_pass = _make_pass()


def _combine_divide(sum_p, cnt_t):
    def body(s_ref, c_ref, o_ref):
        ssum = s_ref[0] + s_ref[1]
        cnt = jnp.sum(c_ref[...], axis=1, keepdims=True)
        o_ref[...] = ssum / jnp.maximum(cnt, 1.0)

    return pl.pallas_call(
        body, out_shape=jax.ShapeDtypeStruct((SEG, D), jnp.float32)
    )(sum_p, cnt_t)


def _finish(sum_p, cnt_t, w_mat, b_row):
    def body(s_ref, c_ref, w_ref, b_ref, o_ref):
        ssum = s_ref[0] + s_ref[1]
        cnt = jnp.sum(c_ref[...], axis=1, keepdims=True)
        h = ssum / jnp.maximum(cnt, 1.0)
        o_ref[...] = lax.dot_general(
            h[:N_NODES], w_ref[...],
            dimension_numbers=(((1,), (1,)), ((), ())),
            preferred_element_type=jnp.float32,
        ) + b_ref[...]

    return pl.pallas_call(
        body, out_shape=jax.ShapeDtypeStruct((N_NODES, D), jnp.float32)
    )(sum_p, cnt_t, w_mat, b_row)


def kernel(features, edge_index, W, b):
    nidx = edge_index[0].astype(jnp.int32)
    hidx = edge_index[1].astype(jnp.int32)
    padv = jnp.full((E_PAD - N_EDGES,), PAD_ROW, jnp.int32)
    nidx_p = jnp.concatenate([nidx, padv])
    hidx_p = jnp.concatenate([hidx, padv])
    feat_p = jnp.zeros((SEG, D), jnp.float32).at[:N_NODES].set(features)

    sum_h, cnt_h = _scatter_mean_pass(feat_p, nidx_p, hidx_p)
    h_edge = _combine_divide(sum_h, cnt_h.T)
    sum_n, cnt_n = _scatter_mean_pass(h_edge, hidx_p, nidx_p)
    return _finish(sum_n, cnt_n.T, W, b.reshape(1, D))


# speedup vs baseline: 3.6888x; 1.2653x over previous
import dataclasses

import jax
import jax.numpy as jnp
from jax import lax
from jax.experimental import pallas as pl
from jax.experimental.pallas import tpu as pltpu
from jax.experimental.pallas import tpu_sc as plsc

N_NODES = 10000
N_HEDGES = 10000
N_EDGES = 320000
D = 128

NC = 2
NS = 16
NW = NC * NS
CHUNK = 128
CPT = 80
E_PAD = NW * CPT * CHUNK
SEG = 10240
PAD_ROW = 10200
ROWS_PER_TILE = SEG // NS

_mesh = plsc.VectorSubcoreMesh(
    core_axis_name="c", subcore_axis_name="s", num_cores=NC, num_subcores=NS
)


def _zero_vmem(ref):
    rows, cols = ref.shape

    @pl.loop(0, rows)
    def _(i):
        for j in range(cols // 16):
            ref[i, pl.ds(j * 16, 16)] = jnp.zeros((16,), ref.dtype)


def _make_pass():
    outs = (
        jax.ShapeDtypeStruct((NC, SEG, D), jnp.float32),
        jax.ShapeDtypeStruct((NW, SEG), jnp.float32),
    )
    scratch = [
        pltpu.VMEM_SHARED((SEG, D), jnp.float32),
        pltpu.VMEM((CHUNK, D), jnp.float32),
        pltpu.VMEM((CHUNK, D), jnp.float32),
        pltpu.VMEM((SEG,), jnp.float32),
        pltpu.VMEM((CHUNK,), jnp.int32),
        pltpu.VMEM((CHUNK,), jnp.int32),
        pltpu.VMEM((CHUNK,), jnp.int32),
        pltpu.VMEM((CHUNK,), jnp.int32),
    ] + [pltpu.SemaphoreType.DMA] * 6

    def body(table_hbm, gidx_hbm, sidx_hbm, sum_out, cnt_out,
             sum_sh, rows_a, rows_b, hist_v, gbuf_a, sbuf_a,
             gbuf_b, sbuf_b, gsem_a, gsem_b, ssem_a, ssem_b,
             isem_a, isem_b):
        c = lax.axis_index("c")
        s = lax.axis_index("s")
        w = c * NS + s
        slot_a = (rows_a, gbuf_a, sbuf_a, gsem_a, ssem_a, isem_a)
        slot_b = (rows_b, gbuf_b, sbuf_b, gsem_b, ssem_b, isem_b)

        _zero_vmem(rows_a)
        rbase = s * ROWS_PER_TILE
        for t in range(ROWS_PER_TILE // CHUNK):
            pltpu.sync_copy(rows_a, sum_sh.at[pl.ds(rbase + t * CHUNK, CHUNK)])

        @pl.loop(0, SEG // 16)
        def _(i):
            hist_v[pl.ds(i * 16, 16)] = jnp.zeros((16,), jnp.float32)

        plsc.subcore_barrier()

        cbase = w * CPT * CHUNK

        def idx_start(k, slot):
            _, gbuf, sbuf, _, _, isem = slot
            eb = cbase + k * CHUNK
            pltpu.async_copy(gidx_hbm.at[pl.ds(eb, CHUNK)], gbuf, isem)
            pltpu.async_copy(sidx_hbm.at[pl.ds(eb, CHUNK)], sbuf, isem)

        def idx_wait(k, slot):
            _, gbuf, sbuf, _, _, isem = slot
            eb = cbase + k * CHUNK
            pltpu.make_async_copy(
                gidx_hbm.at[pl.ds(eb, CHUNK)], gbuf, isem).wait()
            pltpu.make_async_copy(
                sidx_hbm.at[pl.ds(eb, CHUNK)], sbuf, isem).wait()

        def hist_update(sbuf):
            @pl.loop(0, CHUNK // 16)
            def _(q):
                iv = sbuf[pl.ds(q * 16, 16)]
                lane = lax.iota(jnp.int32, 16)
                counts = jnp.zeros((16,), jnp.float32)
                firstpos = jnp.full((16,), 16, jnp.int32)
                for j in range(15, -1, -1):
                    eq = iv == iv[j]
                    counts = counts + jnp.where(eq, 1.0, 0.0)
                    firstpos = jnp.where(eq, j, firstpos)
                head = firstpos == lane
                plsc.addupdate_scatter(hist_v, [iv], counts, mask=head)

        def chunk_body(k, cur, nxt, issue_next_gather, issue_next_idx):
            rows, gbuf, sbuf, gsem, ssem, _ = cur
            if issue_next_gather:
                idx_wait(k + 1, nxt)
                pltpu.async_copy(table_hbm.at[nxt[1]], nxt[0], nxt[3])
            pltpu.make_async_copy(table_hbm.at[gbuf], rows, gsem).wait()
            pltpu.async_copy(rows, sum_sh.at[sbuf], ssem, add=True)
            hist_update(sbuf)
            pltpu.make_async_copy(rows, sum_sh.at[sbuf], ssem).wait()
            if issue_next_idx:
                idx_start(k + 2, cur)

        pltpu.sync_copy(gidx_hbm.at[pl.ds(cbase, CHUNK)], gbuf_a)
        pltpu.sync_copy(sidx_hbm.at[pl.ds(cbase, CHUNK)], sbuf_a)
        idx_start(1, slot_b)
        pltpu.async_copy(table_hbm.at[gbuf_a], rows_a, gsem_a)

        @pl.loop(0, (CPT - 2) // 2)
        def _(p):
            k = 2 * p
            chunk_body(k, slot_a, slot_b, True, True)
            chunk_body(k + 1, slot_b, slot_a, True, True)

        chunk_body(CPT - 2, slot_a, slot_b, True, False)
        chunk_body(CPT - 1, slot_b, slot_a, False, False)

        plsc.subcore_barrier()

        for t in range(ROWS_PER_TILE // CHUNK):
            sl = pl.ds(rbase + t * CHUNK, CHUNK)
            pltpu.sync_copy(sum_sh.at[sl], rows_a)
            pltpu.sync_copy(rows_a, sum_out.at[c, sl])

        pltpu.sync_copy(hist_v, cnt_out.at[w])

    cp = pltpu.CompilerParams()
    if "needs_layout_passes" in pltpu.CompilerParams.__dataclass_fields__:
        cp = dataclasses.replace(cp, needs_layout_passes=False)
    return pl.kernel(
        body, out_type=outs, mesh=_mesh, scratch_types=scratch,
        compiler_params=cp)


_scatter_mean_pass = _make_pass()


def _combine_divide(sum_p, cnt_t):
    def body(s_ref, c_ref, o_ref):
        ssum = s_ref[0] + s_ref[1]
        cnt = jnp.sum(c_ref[...], axis=1, keepdims=True)
        o_ref[...] = ssum / jnp.maximum(cnt, 1.0)

    return pl.pallas_call(
        body, out_shape=jax.ShapeDtypeStruct((SEG, D), jnp.float32)
    )(sum_p, cnt_t)


def _finish(sum_p, cnt_t, w_mat, b_row):
    def body(s_ref, c_ref, w_ref, b_ref, o_ref):
        ssum = s_ref[0] + s_ref[1]
        cnt = jnp.sum(c_ref[...], axis=1, keepdims=True)
        h = ssum / jnp.maximum(cnt, 1.0)
        o_ref[...] = lax.dot_general(
            h[:N_NODES], w_ref[...],
            dimension_numbers=(((1,), (1,)), ((), ())),
            preferred_element_type=jnp.float32,
        ) + b_ref[...]

    return pl.pallas_call(
        body, out_shape=jax.ShapeDtypeStruct((N_NODES, D), jnp.float32)
    )(sum_p, cnt_t, w_mat, b_row)


def kernel(features, edge_index, W, b):
    nidx = edge_index[0].astype(jnp.int32)
    hidx = edge_index[1].astype(jnp.int32)
    padv = jnp.full((E_PAD - N_EDGES,), PAD_ROW, jnp.int32)
    nidx_p = jnp.concatenate([nidx, padv])
    hidx_p = jnp.concatenate([hidx, padv])
    feat_p = jnp.zeros((SEG, D), jnp.float32).at[:N_NODES].set(features)

    sum_h, cnt_h = _scatter_mean_pass(feat_p, nidx_p, hidx_p)
    h_edge = _combine_divide(sum_h, cnt_h.T)
    sum_n, cnt_n = _scatter_mean_pass(h_edge, hidx_p, nidx_p)
    return _finish(sum_n, cnt_n.T, W, b.reshape(1, D))
